# Initial kernel scaffold; baseline (speedup 1.0000x reference)
import jax
import jax.numpy as jnp
from jax.experimental import pallas as pl


def kernel(x, fadj, W1, W2, W3, W4):
    raise NotImplementedError("write your pallas kernel here")


# speedup vs baseline: 1.0645x; 1.0645x over previous
import jax
import jax.numpy as jnp
from jax.experimental import pallas as pl
from jax.experimental.pallas import tpu as pltpu

N = 10000
BM1 = 200
BM = 400
EPS = 1e-5


def _act(o):
    return jnp.where(o >= 0, o, jnp.exp(jnp.minimum(0.2 * o, 0.0)) - 1.0)


def _bdot(a, b):
    return jnp.dot(a, b, preferred_element_type=jnp.float32)


def _layer1_kernel(fadj_ref, s_ref, fb_ref, a_ref, stats_ref):
    i = pl.program_id(0)
    fb = fadj_ref[...].astype(jnp.bfloat16)
    fb_ref[...] = fb
    a = _act(_bdot(fb, s_ref[...]))
    a_ref[...] = a

    @pl.when(i == 0)
    def _():
        stats_ref[...] = jnp.zeros_like(stats_ref)

    stats_ref[0:1, :] += jnp.sum(a, axis=0, keepdims=True)
    stats_ref[1:2, :] += jnp.sum(a * a, axis=0, keepdims=True)


def _layer1(fadj, s):
    h = s.shape[1]
    fb, a, stats = pl.pallas_call(
        _layer1_kernel,
        grid=(N // BM1,),
        in_specs=[
            pl.BlockSpec((BM1, N), lambda i: (i, 0)),
            pl.BlockSpec((N, h), lambda i: (0, 0)),
        ],
        out_specs=[
            pl.BlockSpec((BM1, N), lambda i: (i, 0)),
            pl.BlockSpec((BM1, h), lambda i: (i, 0)),
            pl.BlockSpec((8, h), lambda i: (0, 0)),
        ],
        out_shape=[
            jax.ShapeDtypeStruct((N, N), jnp.bfloat16),
            jax.ShapeDtypeStruct((N, h), jnp.float32),
            jax.ShapeDtypeStruct((8, h), jnp.float32),
        ],
        compiler_params=pltpu.CompilerParams(
            dimension_semantics=("arbitrary",),
        ),
    )(fadj, s)
    return fb, a, stats


def _spmm_act_kernel(fb_ref, s_ref, a_ref, stats_ref):
    i = pl.program_id(0)
    a = _act(_bdot(fb_ref[...], s_ref[...]))
    a_ref[...] = a

    @pl.when(i == 0)
    def _():
        stats_ref[...] = jnp.zeros_like(stats_ref)

    stats_ref[0:1, :] += jnp.sum(a, axis=0, keepdims=True)
    stats_ref[1:2, :] += jnp.sum(a * a, axis=0, keepdims=True)


def _spmm_act(fb, s):
    h = s.shape[1]
    return pl.pallas_call(
        _spmm_act_kernel,
        grid=(N // BM,),
        in_specs=[
            pl.BlockSpec((BM, N), lambda i: (i, 0)),
            pl.BlockSpec((N, h), lambda i: (0, 0)),
        ],
        out_specs=[
            pl.BlockSpec((BM, h), lambda i: (i, 0)),
            pl.BlockSpec((8, h), lambda i: (0, 0)),
        ],
        out_shape=[
            jax.ShapeDtypeStruct((N, h), jnp.float32),
            jax.ShapeDtypeStruct((8, h), jnp.float32),
        ],
        compiler_params=pltpu.CompilerParams(
            dimension_semantics=("arbitrary",),
        ),
    )(fb, s)


def _bn_mm_kernel(a_ref, stats_ref, w_ref, h_ref, s_ref):
    m = stats_ref[0:1, :] / N
    v = stats_ref[1:2, :] / N - m * m
    hmat = (a_ref[...] - m) * jax.lax.rsqrt(v + EPS)
    h_ref[...] = hmat
    s_ref[...] = _bdot(hmat.astype(jnp.bfloat16), w_ref[...].astype(jnp.bfloat16)).astype(jnp.bfloat16)


def _bn_mm(a, stats, w):
    h = a.shape[1]
    hn = w.shape[1]
    return pl.pallas_call(
        _bn_mm_kernel,
        out_shape=[
            jax.ShapeDtypeStruct((N, h), jnp.float32),
            jax.ShapeDtypeStruct((N, hn), jnp.bfloat16),
        ],
    )(a, stats, w)


def _bn_kernel(a_ref, stats_ref, h_ref):
    m = stats_ref[0:1, :] / N
    v = stats_ref[1:2, :] / N - m * m
    h_ref[...] = (a_ref[...] - m) * jax.lax.rsqrt(v + EPS)


def _bn(a, stats):
    h = a.shape[1]
    return pl.pallas_call(
        _bn_kernel,
        out_shape=jax.ShapeDtypeStruct((N, h), jnp.float32),
    )(a, stats)


def _mm_kernel(x_ref, w_ref, s_ref):
    s_ref[...] = _bdot(
        x_ref[...].astype(jnp.bfloat16), w_ref[...].astype(jnp.bfloat16)
    ).astype(jnp.bfloat16)


def _mm(x, w):
    return pl.pallas_call(
        _mm_kernel,
        out_shape=jax.ShapeDtypeStruct((x.shape[0], w.shape[1]), jnp.bfloat16),
    )(x, w)


def kernel(x, fadj, W1, W2, W3, W4):
    s1 = _mm(x, W1)
    fb, a1, st1 = _layer1(fadj, s1)
    h1, s2 = _bn_mm(a1, st1, W2)
    a2, st2 = _spmm_act(fb, s2)
    h2, s3 = _bn_mm(a2, st2, W3)
    a3, st3 = _spmm_act(fb, s3)
    h3, s4 = _bn_mm(a3, st3, W4)
    a4, st4 = _spmm_act(fb, s4)
    h4 = _bn(a4, st4)
    return (h1, h2, h3, h4)
